# baseline (speedup 1.0000x reference)
import functools

import jax
import jax.numpy as jnp
from jax import lax
from jax.experimental import pallas as pl
from jax.experimental.pallas import tpu as pltpu
from jax.experimental.pallas import tpu_sc as plsc

_D = 64
_NC, _NS = 2, 16
_NW = _NC * _NS
_TB = 128
_SW = 128


@functools.cache
def _make_transpose(V):
    n_slab = V // _SW
    tail = V % _SW
    base, rem = divmod(n_slab, _NW)
    trip = base + (1 if rem else 0)
    assert tail % 16 == 0 and tail <= 128 and _D == 64

    def body(w3_hbm, wt_hbm, flat_hbm, slab, wbuf0, wbuf1, rsem, wsem):
        wid = lax.axis_index("s") * _NC + lax.axis_index("c")
        n_w = base + jnp.where(wid < rem, 1, 0)
        start = wid * base + jnp.minimum(wid, rem)
        iota = lax.iota(jnp.int32, 16)
        iota64 = iota * _D
        wbufs = [wbuf0, wbuf1]

        def r_start(k, b):
            j0 = (start + k) * _SW
            pltpu.async_copy(w3_hbm.at[:, pl.ds(j0, _SW)], slab.at[b],
                             rsem.at[b])

        def r_wait(k, b):
            j0 = (start + k) * _SW
            pltpu.make_async_copy(w3_hbm.at[:, pl.ds(j0, _SW)], slab.at[b],
                                  rsem.at[b]).wait()

        def tp(sb, db):
            @plsc.parallel_loop(0, (_SW // 16) * _D, unroll=8)
            def _(i):
                m = lax.shift_right_logical(i, 6)
                c = lax.bitwise_and(i, _D - 1)
                v = slab[sb, c, pl.ds(m * 16, 16)]
                plsc.store_scatter(wbufs[db],
                                   [iota64 + (m * 16 * _D + c)], v)

        def w_start(k, b):
            j0 = (start + k) * _SW
            pltpu.async_copy(wbufs[b], flat_hbm.at[pl.ds(j0 * _D, _SW * _D)],
                             wsem.at[b])

        def w_wait(k, b):
            j0 = (start + k) * _SW
            pltpu.make_async_copy(
                wbufs[b], flat_hbm.at[pl.ds(j0 * _D, _SW * _D)],
                wsem.at[b]).wait()

        r_start(0, 0)

        def blk(i, carry):
            for u in range(2):
                k = i * 2 + u

                @pl.when(k + 1 < n_w)
                def _():
                    r_start(k + 1, (u + 1) % 2)

                @pl.when(k < n_w)
                def _():
                    r_wait(k, u)

                    @pl.when(k >= 2)
                    def _():
                        w_wait(k - 2, u)

                    tp(u, u)
                    w_start(k, u)
            return carry

        lax.fori_loop(0, (trip + 1) // 2, blk, 0)

        @pl.when(n_w % 2 == 0)
        def _():
            w_wait(n_w - 2, 0)
            w_wait(n_w - 1, 1)

        @pl.when(n_w % 2 == 1)
        def _():
            w_wait(n_w - 2, 1)
            w_wait(n_w - 1, 0)

        if tail:
            @pl.when(wid == _NW - 1)
            def _():
                j0 = n_slab * _SW
                pltpu.sync_copy(wt_hbm, slab.at[0])
                for m in range(tail // 16):
                    for c in range(_D):
                        v = slab[0, c, pl.ds(m * 16, 16)]
                        plsc.store_scatter(wbuf0,
                                           [iota64 + (m * 16 * _D + c)], v)
                pltpu.sync_copy(wbuf0.at[pl.ds(0, tail * _D)],
                                flat_hbm.at[pl.ds(j0 * _D, tail * _D)])

    return pl.kernel(
        body,
        mesh=plsc.VectorSubcoreMesh(core_axis_name="c", subcore_axis_name="s"),
        compiler_params=pltpu.CompilerParams(use_tc_tiling_on_sc=True,
                                             needs_layout_passes=False),
        out_type=jax.ShapeDtypeStruct((V * _D,), jnp.float32),
        scratch_types=[
            pltpu.VMEM((2, _D, _SW), jnp.float32),
            pltpu.VMEM((_SW * _D,), jnp.float32),
            pltpu.VMEM((_SW * _D,), jnp.float32),
            pltpu.SemaphoreType.DMA((2,)),
            pltpu.SemaphoreType.DMA((2,)),
        ],
    )


@functools.cache
def _make_lookup(T, R):
    t_per_w = T // _NW
    nb_t = t_per_w // _TB
    n = R * nb_t
    assert T % (_NW * _TB) == 0 and nb_t == 4 and R % 2 == 0

    def body(idx_hbm, table_hbm, out_hbm, idx_v, idxl, rows, outb, gsem, wsem):
        wid = lax.axis_index("s") * _NC + lax.axis_index("c")
        tw0 = wid * t_per_w
        iota = lax.iota(jnp.int32, 16)
        zeros16 = jnp.zeros((16,), jnp.int32)
        ciota = [c0 + iota for c0 in range(0, _D, 16)]

        pltpu.sync_copy(idx_hbm.at[pl.ds(tw0 * R, t_per_w * R)], idx_v)

        def build_list(b, buf):
            r = b // nb_t
            tb = b % nb_t
            for k in range(_TB // 16):
                src = (tb * _TB + k * 16 + iota) * R + r
                idxl[buf, pl.ds(k * 16, 16)] = plsc.load_gather(idx_v, [src])

        def g_start(buf):
            pltpu.async_copy(table_hbm.at[idxl.at[buf]], rows.at[buf],
                             gsem.at[buf])

        def g_wait(buf):
            pltpu.make_async_copy(table_hbm.at[idxl.at[buf]], rows.at[buf],
                                  gsem.at[buf]).wait()

        def out_slice(r):
            return out_hbm.at[r, :, pl.ds(tw0, t_per_w)]

        def w_start(r, buf):
            pltpu.async_copy(outb.at[buf], out_slice(r), wsem.at[buf])

        def w_wait(r, buf):
            pltpu.make_async_copy(outb.at[buf], out_slice(r),
                                  wsem.at[buf]).wait()

        def tp_block(sbuf, dbuf, tb):
            @plsc.parallel_loop(0, _TB, unroll=4)
            def _(t):
                tv = (t + tb * _TB) + zeros16
                for j in range(_D // 16):
                    v = rows[sbuf, t, pl.ds(j * 16, 16)]
                    plsc.store_scatter(outb.at[dbuf], [ciota[j], tv], v)

        for b in range(3):
            build_list(b, b)
            g_start(b)

        def blk(i, carry):
            for rr in range(2):
                r = i * 2 + rr

                @pl.when(r >= 2)
                def _():
                    w_wait(r - 2, rr)

                for tb in range(nb_t):
                    b = r * nb_t + tb
                    nxt = b + 3

                    @pl.when(nxt < n)
                    def _():
                        build_list(nxt, (tb + 3) % 4)
                        g_start((tb + 3) % 4)

                    g_wait(tb)
                    tp_block(tb, rr, tb)
                w_start(r, rr)
            return carry

        lax.fori_loop(0, R // 2, blk, 0)
        w_wait(R - 2, 0)
        w_wait(R - 1, 1)

    return pl.kernel(
        body,
        mesh=plsc.VectorSubcoreMesh(core_axis_name="c", subcore_axis_name="s"),
        compiler_params=pltpu.CompilerParams(use_tc_tiling_on_sc=False,
                                             needs_layout_passes=False),
        out_type=jax.ShapeDtypeStruct((R, _D, T), jnp.float32),
        scratch_types=[
            pltpu.VMEM((T // _NW * R,), jnp.int32),
            pltpu.VMEM((4, _TB), jnp.int32),
            pltpu.VMEM((4, _TB, _D), jnp.float32),
            pltpu.VMEM((2, _D, T // _NW), jnp.float32),
            pltpu.SemaphoreType.DMA((4,)),
            pltpu.SemaphoreType.DMA((2,)),
        ],
    )


def kernel(token_ids, weight):
    T, R = token_ids.shape
    V = weight.shape[0]
    flat_ids = token_ids.reshape(-1).astype(jnp.int32)
    v_full = (V // _SW) * _SW
    wt_pad = jnp.pad(weight[v_full:].T, ((0, 0), (0, 128 - (V - v_full))))
    table_flat = _make_transpose(V)(weight.T, wt_pad)
    out3 = _make_lookup(T, R)(flat_ids, table_flat.reshape(V, _D))
    return jnp.transpose(out3, (2, 0, 1))


# speedup vs baseline: 2.6969x; 2.6969x over previous
import functools

import jax
import jax.numpy as jnp
from jax import lax
from jax.experimental import pallas as pl
from jax.experimental.pallas import tpu as pltpu
from jax.experimental.pallas import tpu_sc as plsc

_D = 64
_NC, _NS = 2, 16
_NW = _NC * _NS
_TB = 128
_SW = 128


@functools.cache
def _make_transpose(V):
    n_slab = V // _SW
    tail = V % _SW
    base, rem = divmod(n_slab, _NW)
    trip = base + (1 if rem else 0)
    assert tail % 16 == 0 and tail <= 128 and _D == 64

    def body(w3_hbm, wt_hbm, flat_hbm, slab, wbuf0, wbuf1, wpad, rsem, wsem):
        wid = lax.axis_index("s") * _NC + lax.axis_index("c")
        n_w = base + jnp.where(wid < rem, 1, 0)
        start = wid * base + jnp.minimum(wid, rem)
        iota = lax.iota(jnp.int32, 16)
        iota65 = iota * 65
        wbufs = [wbuf0, wbuf1]

        def r_start(k, b):
            j0 = (start + k) * _SW
            pltpu.async_copy(w3_hbm.at[:, pl.ds(j0, _SW)], slab.at[b],
                             rsem.at[b])

        def r_wait(k, b):
            j0 = (start + k) * _SW
            pltpu.make_async_copy(w3_hbm.at[:, pl.ds(j0, _SW)], slab.at[b],
                                  rsem.at[b]).wait()

        def tp(sb, db):
            @plsc.parallel_loop(0, (_SW // 16) * _D, unroll=8)
            def _(i):
                m = lax.shift_right_logical(i, 6)
                c = lax.bitwise_and(i, _D - 1)
                v = slab[sb, c, pl.ds(m * 16, 16)]
                plsc.store_scatter(wpad, [iota65 + (m * 16 * 65 + c)], v)

            @plsc.parallel_loop(0, _SW * (_D // 16), unroll=8)
            def _(i):
                j = lax.shift_right_logical(i, 2)
                c0 = lax.bitwise_and(i, 3) * 16
                wbufs[db][pl.ds(j * _D + c0, 16)] = wpad[pl.ds(j * 65 + c0,
                                                              16)]

        def w_start(k, b):
            j0 = (start + k) * _SW
            pltpu.async_copy(wbufs[b], flat_hbm.at[pl.ds(j0 * _D, _SW * _D)],
                             wsem.at[b])

        def w_wait(k, b):
            j0 = (start + k) * _SW
            pltpu.make_async_copy(
                wbufs[b], flat_hbm.at[pl.ds(j0 * _D, _SW * _D)],
                wsem.at[b]).wait()

        r_start(0, 0)

        def blk(i, carry):
            for u in range(2):
                k = i * 2 + u

                @pl.when(k + 1 < n_w)
                def _():
                    r_start(k + 1, (u + 1) % 2)

                @pl.when(k < n_w)
                def _():
                    r_wait(k, u)

                    @pl.when(k >= 2)
                    def _():
                        w_wait(k - 2, u)

                    tp(u, u)
                    w_start(k, u)
            return carry

        lax.fori_loop(0, (trip + 1) // 2, blk, 0)

        @pl.when(n_w % 2 == 0)
        def _():
            w_wait(n_w - 2, 0)
            w_wait(n_w - 1, 1)

        @pl.when(n_w % 2 == 1)
        def _():
            w_wait(n_w - 2, 1)
            w_wait(n_w - 1, 0)

        if tail:
            @pl.when(wid == _NW - 1)
            def _():
                j0 = n_slab * _SW
                pltpu.sync_copy(wt_hbm, slab.at[0])
                for m in range(tail // 16):
                    for c in range(_D):
                        v = slab[0, c, pl.ds(m * 16, 16)]
                        plsc.store_scatter(wpad,
                                           [iota65 + (m * 16 * 65 + c)], v)
                for i in range(tail * (_D // 16)):
                    j, c0 = i // 4, (i % 4) * 16
                    wbuf0[pl.ds(j * _D + c0, 16)] = wpad[pl.ds(j * 65 + c0,
                                                               16)]
                pltpu.sync_copy(wbuf0.at[pl.ds(0, tail * _D)],
                                flat_hbm.at[pl.ds(j0 * _D, tail * _D)])

    return pl.kernel(
        body,
        mesh=plsc.VectorSubcoreMesh(core_axis_name="c", subcore_axis_name="s"),
        compiler_params=pltpu.CompilerParams(use_tc_tiling_on_sc=True,
                                             needs_layout_passes=False),
        out_type=jax.ShapeDtypeStruct((V * _D,), jnp.float32),
        scratch_types=[
            pltpu.VMEM((2, _D, _SW), jnp.float32),
            pltpu.VMEM((_SW * _D,), jnp.float32),
            pltpu.VMEM((_SW * _D,), jnp.float32),
            pltpu.VMEM((_SW * 65,), jnp.float32),
            pltpu.SemaphoreType.DMA((2,)),
            pltpu.SemaphoreType.DMA((2,)),
        ],
    )


@functools.cache
def _make_lookup(T, R):
    t_per_w = T // _NW
    nb_t = t_per_w // _TB
    n = R * nb_t
    assert T % (_NW * _TB) == 0 and nb_t == 4 and R % 2 == 0

    def body(idx_hbm, table_hbm, out_hbm, idx_v, idxl, rows, outb, opad,
             gsem, wsem):
        wid = lax.axis_index("s") * _NC + lax.axis_index("c")
        tw0 = wid * t_per_w
        iota = lax.iota(jnp.int32, 16)
        zeros16 = jnp.zeros((16,), jnp.int32)
        ciota513 = [(c0 + iota) * 513 for c0 in range(0, _D, 16)]

        pltpu.sync_copy(idx_hbm.at[pl.ds(tw0 * R, t_per_w * R)], idx_v)

        def build_list(b, buf):
            r = b // nb_t
            tb = b % nb_t
            for k in range(_TB // 16):
                src = (tb * _TB + k * 16 + iota) * R + r
                idxl[buf, pl.ds(k * 16, 16)] = plsc.load_gather(idx_v, [src])

        def g_start(buf):
            pltpu.async_copy(table_hbm.at[idxl.at[buf]], rows.at[buf],
                             gsem.at[buf])

        def g_wait(buf):
            pltpu.make_async_copy(table_hbm.at[idxl.at[buf]], rows.at[buf],
                                  gsem.at[buf]).wait()

        def out_slice(r):
            return out_hbm.at[r, :, pl.ds(tw0, t_per_w)]

        def w_start(r):
            pltpu.async_copy(outb, out_slice(r), wsem)

        def w_wait(r):
            pltpu.make_async_copy(outb, out_slice(r), wsem).wait()

        def tp_block(sbuf, tb):
            @plsc.parallel_loop(0, _TB, unroll=4)
            def _(t):
                tv = (t + tb * _TB) + zeros16
                for j in range(_D // 16):
                    v = rows[sbuf, t, pl.ds(j * 16, 16)]
                    plsc.store_scatter(opad, [ciota513[j] + tv], v)

        def pack():
            @plsc.parallel_loop(0, _D * (t_per_w // 16), unroll=8)
            def _(i):
                c = lax.shift_right_logical(i, 5)
                t0 = lax.bitwise_and(i, 31) * 16
                outb[c, pl.ds(t0, 16)] = opad[pl.ds(c * 513 + t0, 16)]

        for b in range(3):
            build_list(b, b)
            g_start(b)

        def blk(r, carry):
            for tb in range(nb_t):
                b = r * nb_t + tb
                nxt = b + 3

                @pl.when(nxt < n)
                def _():
                    build_list(nxt, (tb + 3) % 4)
                    g_start((tb + 3) % 4)

                g_wait(tb)
                tp_block(tb, tb)

            @pl.when(r >= 1)
            def _():
                w_wait(r - 1)

            pack()
            w_start(r)
            return carry

        lax.fori_loop(0, R, blk, 0)
        w_wait(R - 1)

    return pl.kernel(
        body,
        mesh=plsc.VectorSubcoreMesh(core_axis_name="c", subcore_axis_name="s"),
        compiler_params=pltpu.CompilerParams(use_tc_tiling_on_sc=False,
                                             needs_layout_passes=False),
        out_type=jax.ShapeDtypeStruct((R, _D, T), jnp.float32),
        scratch_types=[
            pltpu.VMEM((T // _NW * R,), jnp.int32),
            pltpu.VMEM((4, _TB), jnp.int32),
            pltpu.VMEM((4, _TB, _D), jnp.float32),
            pltpu.VMEM((_D, T // _NW), jnp.float32),
            pltpu.VMEM((_D * 513,), jnp.float32),
            pltpu.SemaphoreType.DMA((4,)),
            pltpu.SemaphoreType.DMA,
        ],
    )


def kernel(token_ids, weight):
    T, R = token_ids.shape
    V = weight.shape[0]
    flat_ids = token_ids.reshape(-1).astype(jnp.int32)
    v_full = (V // _SW) * _SW
    wt_pad = jnp.pad(weight[v_full:].T, ((0, 0), (0, 128 - (V - v_full))))
    table_flat = _make_transpose(V)(weight.T, wt_pad)
    out3 = _make_lookup(T, R)(flat_ids, table_flat.reshape(V, _D))
    return jnp.transpose(out3, (2, 0, 1))


# speedup vs baseline: 2.9963x; 1.1110x over previous
import functools

import jax
import jax.numpy as jnp
from jax import lax
from jax.experimental import pallas as pl
from jax.experimental.pallas import tpu as pltpu
from jax.experimental.pallas import tpu_sc as plsc

_D = 64
_NC, _NS = 2, 16
_NW = _NC * _NS
_TB = 128
_SW = 256


@functools.cache
def _make_transpose(V):
    n_slab = V // _SW
    tail = V % _SW
    base, rem = divmod(n_slab, _NW)
    trip = base + (1 if rem else 0)
    assert tail % 16 == 0 and tail <= 128 and _D == 64

    def body(w3_hbm, wt_hbm, flat_hbm, slab, wbuf0, wbuf1, wpad, rsem, wsem):
        wid = lax.axis_index("s") * _NC + lax.axis_index("c")
        n_w = base + jnp.where(wid < rem, 1, 0)
        start = wid * base + jnp.minimum(wid, rem)
        iota = lax.iota(jnp.int32, 16)
        iota65 = iota * 65
        wbufs = [wbuf0, wbuf1]

        def r_start(k, b):
            j0 = (start + k) * _SW
            pltpu.async_copy(w3_hbm.at[:, pl.ds(j0, _SW)], slab.at[b],
                             rsem.at[b])

        def r_wait(k, b):
            j0 = (start + k) * _SW
            pltpu.make_async_copy(w3_hbm.at[:, pl.ds(j0, _SW)], slab.at[b],
                                  rsem.at[b]).wait()

        def tp(sb, db):
            @plsc.parallel_loop(0, (_SW // 16) * _D, unroll=8)
            def _(i):
                m = lax.shift_right_logical(i, 6)
                c = lax.bitwise_and(i, _D - 1)
                v = slab[sb, c, pl.ds(m * 16, 16)]
                plsc.store_scatter(wpad, [iota65 + (m * 16 * 65 + c)], v)

            @plsc.parallel_loop(0, _SW * (_D // 16), unroll=8)
            def _(i):
                j = lax.shift_right_logical(i, 2)
                c0 = lax.bitwise_and(i, 3) * 16
                wbufs[db][pl.ds(j * _D + c0, 16)] = wpad[pl.ds(j * 65 + c0,
                                                              16)]

        def w_start(k, b):
            j0 = (start + k) * _SW
            pltpu.async_copy(wbufs[b], flat_hbm.at[pl.ds(j0 * _D, _SW * _D)],
                             wsem.at[b])

        def w_wait(k, b):
            j0 = (start + k) * _SW
            pltpu.make_async_copy(
                wbufs[b], flat_hbm.at[pl.ds(j0 * _D, _SW * _D)],
                wsem.at[b]).wait()

        r_start(0, 0)

        def blk(i, carry):
            for u in range(2):
                k = i * 2 + u

                @pl.when(k + 1 < n_w)
                def _():
                    r_start(k + 1, (u + 1) % 2)

                @pl.when(k < n_w)
                def _():
                    r_wait(k, u)

                    @pl.when(k >= 2)
                    def _():
                        w_wait(k - 2, u)

                    tp(u, u)
                    w_start(k, u)
            return carry

        lax.fori_loop(0, (trip + 1) // 2, blk, 0)

        @pl.when(n_w % 2 == 0)
        def _():
            w_wait(n_w - 2, 0)
            w_wait(n_w - 1, 1)

        @pl.when(n_w % 2 == 1)
        def _():
            w_wait(n_w - 2, 1)
            w_wait(n_w - 1, 0)

        if tail:
            @pl.when(wid == _NW - 1)
            def _():
                j0 = n_slab * _SW
                pltpu.sync_copy(wt_hbm, slab.at[0, :, pl.ds(0, 128)])
                for m in range(tail // 16):
                    for c in range(_D):
                        v = slab[0, c, pl.ds(m * 16, 16)]
                        plsc.store_scatter(wpad,
                                           [iota65 + (m * 16 * 65 + c)], v)
                for i in range(tail * (_D // 16)):
                    j, c0 = i // 4, (i % 4) * 16
                    wbuf0[pl.ds(j * _D + c0, 16)] = wpad[pl.ds(j * 65 + c0,
                                                               16)]
                pltpu.sync_copy(wbuf0.at[pl.ds(0, tail * _D)],
                                flat_hbm.at[pl.ds(j0 * _D, tail * _D)])

    return pl.kernel(
        body,
        mesh=plsc.VectorSubcoreMesh(core_axis_name="c", subcore_axis_name="s"),
        compiler_params=pltpu.CompilerParams(use_tc_tiling_on_sc=True,
                                             needs_layout_passes=False),
        out_type=jax.ShapeDtypeStruct((V * _D,), jnp.float32),
        scratch_types=[
            pltpu.VMEM((2, _D, _SW), jnp.float32),
            pltpu.VMEM((_SW * _D,), jnp.float32),
            pltpu.VMEM((_SW * _D,), jnp.float32),
            pltpu.VMEM((_SW * 65,), jnp.float32),
            pltpu.SemaphoreType.DMA((2,)),
            pltpu.SemaphoreType.DMA((2,)),
        ],
    )


@functools.cache
def _make_lookup(T, R):
    t_per_w = T // _NW
    nb_t = t_per_w // _TB
    n = R * nb_t
    assert T % (_NW * _TB) == 0 and nb_t == 4 and R % 2 == 0

    def body(idx_hbm, table_hbm, out_hbm, idx_v, idxl, rows, outb, opad,
             gsem, wsem):
        wid = lax.axis_index("s") * _NC + lax.axis_index("c")
        tw0 = wid * t_per_w
        iota = lax.iota(jnp.int32, 16)
        zeros16 = jnp.zeros((16,), jnp.int32)
        ciota513 = [(c0 + iota) * 513 for c0 in range(0, _D, 16)]

        pltpu.sync_copy(idx_hbm.at[pl.ds(tw0 * R, t_per_w * R)], idx_v)

        def build_list(b, buf):
            r = b // nb_t
            tb = b % nb_t
            for k in range(_TB // 16):
                src = (tb * _TB + k * 16 + iota) * R + r
                idxl[buf, pl.ds(k * 16, 16)] = plsc.load_gather(idx_v, [src])

        def g_start(buf):
            pltpu.async_copy(table_hbm.at[idxl.at[buf]], rows.at[buf],
                             gsem.at[buf])

        def g_wait(buf):
            pltpu.make_async_copy(table_hbm.at[idxl.at[buf]], rows.at[buf],
                                  gsem.at[buf]).wait()

        def out_slice(r):
            return out_hbm.at[r, :, pl.ds(tw0, t_per_w)]

        def w_start(r):
            pltpu.async_copy(outb, out_slice(r), wsem)

        def w_wait(r):
            pltpu.make_async_copy(outb, out_slice(r), wsem).wait()

        def tp_block(sbuf, tb):
            @plsc.parallel_loop(0, _TB, unroll=4)
            def _(t):
                tv = (t + tb * _TB) + zeros16
                for j in range(_D // 16):
                    v = rows[sbuf, t, pl.ds(j * 16, 16)]
                    plsc.store_scatter(opad, [ciota513[j] + tv], v)

        def pack():
            @plsc.parallel_loop(0, _D * (t_per_w // 16), unroll=8)
            def _(i):
                c = lax.shift_right_logical(i, 5)
                t0 = lax.bitwise_and(i, 31) * 16
                outb[c, pl.ds(t0, 16)] = opad[pl.ds(c * 513 + t0, 16)]

        for b in range(3):
            build_list(b, b)
            g_start(b)

        def blk(r, carry):
            for tb in range(nb_t):
                b = r * nb_t + tb
                nxt = b + 3

                @pl.when(nxt < n)
                def _():
                    build_list(nxt, (tb + 3) % 4)
                    g_start((tb + 3) % 4)

                g_wait(tb)
                tp_block(tb, tb)

            @pl.when(r >= 1)
            def _():
                w_wait(r - 1)

            pack()
            w_start(r)
            return carry

        lax.fori_loop(0, R, blk, 0)
        w_wait(R - 1)

    return pl.kernel(
        body,
        mesh=plsc.VectorSubcoreMesh(core_axis_name="c", subcore_axis_name="s"),
        compiler_params=pltpu.CompilerParams(use_tc_tiling_on_sc=False,
                                             needs_layout_passes=False),
        out_type=jax.ShapeDtypeStruct((R, _D, T), jnp.float32),
        scratch_types=[
            pltpu.VMEM((T // _NW * R,), jnp.int32),
            pltpu.VMEM((4, _TB), jnp.int32),
            pltpu.VMEM((4, _TB, _D), jnp.float32),
            pltpu.VMEM((_D, T // _NW), jnp.float32),
            pltpu.VMEM((_D * 513,), jnp.float32),
            pltpu.SemaphoreType.DMA((4,)),
            pltpu.SemaphoreType.DMA,
        ],
    )


def kernel(token_ids, weight):
    T, R = token_ids.shape
    V = weight.shape[0]
    flat_ids = token_ids.reshape(-1).astype(jnp.int32)
    v_full = (V // _SW) * _SW
    wt_pad = jnp.pad(weight[v_full:].T, ((0, 0), (0, 128 - (V - v_full))))
    table_flat = _make_transpose(V)(weight.T, wt_pad)
    out3 = _make_lookup(T, R)(flat_ids, table_flat.reshape(V, _D))
    return jnp.transpose(out3, (2, 0, 1))


# speedup vs baseline: 3.0549x; 1.0196x over previous
import functools

import jax
import jax.numpy as jnp
from jax import lax
from jax.experimental import pallas as pl
from jax.experimental.pallas import tpu as pltpu
from jax.experimental.pallas import tpu_sc as plsc

_D = 64
_NC, _NS = 2, 16
_NW = _NC * _NS
_TB = 128
_SW = 384


@functools.cache
def _make_transpose(V):
    n_slab = V // _SW
    tail = V % _SW
    base, rem = divmod(n_slab, _NW)
    trip = base + (1 if rem else 0)
    assert tail % 16 == 0 and tail <= 128 and _D == 64

    def body(w3_hbm, wt_hbm, flat_hbm, slab, wbuf0, wbuf1, wpad, rsem, wsem):
        wid = lax.axis_index("s") * _NC + lax.axis_index("c")
        n_w = base + jnp.where(wid < rem, 1, 0)
        start = wid * base + jnp.minimum(wid, rem)
        iota = lax.iota(jnp.int32, 16)
        iota65 = iota * 65
        wbufs = [wbuf0, wbuf1]

        def r_start(k, b):
            j0 = (start + k) * _SW
            pltpu.async_copy(w3_hbm.at[:, pl.ds(j0, _SW)], slab.at[b],
                             rsem.at[b])

        def r_wait(k, b):
            j0 = (start + k) * _SW
            pltpu.make_async_copy(w3_hbm.at[:, pl.ds(j0, _SW)], slab.at[b],
                                  rsem.at[b]).wait()

        def tp(sb, db):
            @plsc.parallel_loop(0, (_SW // 16) * _D, unroll=8)
            def _(i):
                m = lax.shift_right_logical(i, 6)
                c = lax.bitwise_and(i, _D - 1)
                v = slab[sb, c, pl.ds(m * 16, 16)]
                plsc.store_scatter(wpad, [iota65 + (m * 16 * 65 + c)], v)

            @plsc.parallel_loop(0, _SW * (_D // 16), unroll=8)
            def _(i):
                j = lax.shift_right_logical(i, 2)
                c0 = lax.bitwise_and(i, 3) * 16
                wbufs[db][pl.ds(j * _D + c0, 16)] = wpad[pl.ds(j * 65 + c0,
                                                              16)]

        def w_start(k, b):
            j0 = (start + k) * _SW
            pltpu.async_copy(wbufs[b], flat_hbm.at[pl.ds(j0 * _D, _SW * _D)],
                             wsem.at[b])

        def w_wait(k, b):
            j0 = (start + k) * _SW
            pltpu.make_async_copy(
                wbufs[b], flat_hbm.at[pl.ds(j0 * _D, _SW * _D)],
                wsem.at[b]).wait()

        r_start(0, 0)

        def blk(i, carry):
            for u in range(2):
                k = i * 2 + u

                @pl.when(k + 1 < n_w)
                def _():
                    r_start(k + 1, (u + 1) % 2)

                @pl.when(k < n_w)
                def _():
                    r_wait(k, u)

                    @pl.when(k >= 2)
                    def _():
                        w_wait(k - 2, u)

                    tp(u, u)
                    w_start(k, u)
            return carry

        lax.fori_loop(0, (trip + 1) // 2, blk, 0)

        @pl.when(n_w % 2 == 0)
        def _():
            w_wait(n_w - 2, 0)
            w_wait(n_w - 1, 1)

        @pl.when(n_w % 2 == 1)
        def _():
            w_wait(n_w - 2, 1)
            w_wait(n_w - 1, 0)

        if tail:
            @pl.when(wid == _NW - 1)
            def _():
                j0 = n_slab * _SW
                pltpu.sync_copy(wt_hbm, slab.at[0, :, pl.ds(0, 128)])
                for m in range(tail // 16):
                    for c in range(_D):
                        v = slab[0, c, pl.ds(m * 16, 16)]
                        plsc.store_scatter(wpad,
                                           [iota65 + (m * 16 * 65 + c)], v)
                for i in range(tail * (_D // 16)):
                    j, c0 = i // 4, (i % 4) * 16
                    wbuf0[pl.ds(j * _D + c0, 16)] = wpad[pl.ds(j * 65 + c0,
                                                               16)]
                pltpu.sync_copy(wbuf0.at[pl.ds(0, tail * _D)],
                                flat_hbm.at[pl.ds(j0 * _D, tail * _D)])

    return pl.kernel(
        body,
        mesh=plsc.VectorSubcoreMesh(core_axis_name="c", subcore_axis_name="s"),
        compiler_params=pltpu.CompilerParams(use_tc_tiling_on_sc=True,
                                             needs_layout_passes=False),
        out_type=jax.ShapeDtypeStruct((V * _D,), jnp.float32),
        scratch_types=[
            pltpu.VMEM((2, _D, _SW), jnp.float32),
            pltpu.VMEM((_SW * _D,), jnp.float32),
            pltpu.VMEM((_SW * _D,), jnp.float32),
            pltpu.VMEM((_SW * 65,), jnp.float32),
            pltpu.SemaphoreType.DMA((2,)),
            pltpu.SemaphoreType.DMA((2,)),
        ],
    )


@functools.cache
def _make_lookup(T, R):
    t_per_w = T // _NW
    nb_t = t_per_w // _TB
    n = R * nb_t
    assert T % (_NW * _TB) == 0 and nb_t == 4 and R % 2 == 0

    def body(idx_hbm, table_hbm, out_hbm, idx_v, idxl, rows, outb, opad,
             gsem, wsem):
        wid = lax.axis_index("s") * _NC + lax.axis_index("c")
        tw0 = wid * t_per_w
        iota = lax.iota(jnp.int32, 16)
        zeros16 = jnp.zeros((16,), jnp.int32)
        ciota513 = [(c0 + iota) * 513 for c0 in range(0, _D, 16)]

        pltpu.sync_copy(idx_hbm.at[pl.ds(tw0 * R, t_per_w * R)], idx_v)

        def build_list(b, buf):
            r = b // nb_t
            tb = b % nb_t
            for k in range(_TB // 16):
                src = (tb * _TB + k * 16 + iota) * R + r
                idxl[buf, pl.ds(k * 16, 16)] = plsc.load_gather(idx_v, [src])

        def g_start(buf):
            pltpu.async_copy(table_hbm.at[idxl.at[buf]], rows.at[buf],
                             gsem.at[buf])

        def g_wait(buf):
            pltpu.make_async_copy(table_hbm.at[idxl.at[buf]], rows.at[buf],
                                  gsem.at[buf]).wait()

        def out_slice(r):
            return out_hbm.at[r, :, pl.ds(tw0, t_per_w)]

        def w_start(r):
            pltpu.async_copy(outb, out_slice(r), wsem)

        def w_wait(r):
            pltpu.make_async_copy(outb, out_slice(r), wsem).wait()

        def tp_block(sbuf, tb):
            @plsc.parallel_loop(0, _TB, unroll=4)
            def _(t):
                tv = (t + tb * _TB) + zeros16
                for j in range(_D // 16):
                    v = rows[sbuf, t, pl.ds(j * 16, 16)]
                    plsc.store_scatter(opad, [ciota513[j] + tv], v)

        def pack():
            @plsc.parallel_loop(0, _D * (t_per_w // 16), unroll=8)
            def _(i):
                c = lax.shift_right_logical(i, 5)
                t0 = lax.bitwise_and(i, 31) * 16
                outb[c, pl.ds(t0, 16)] = opad[pl.ds(c * 513 + t0, 16)]

        for b in range(3):
            build_list(b, b)
            g_start(b)

        def blk(r, carry):
            for tb in range(nb_t):
                b = r * nb_t + tb
                nxt = b + 3

                @pl.when(nxt < n)
                def _():
                    build_list(nxt, (tb + 3) % 4)
                    g_start((tb + 3) % 4)

                g_wait(tb)
                tp_block(tb, tb)

            @pl.when(r >= 1)
            def _():
                w_wait(r - 1)

            pack()
            w_start(r)
            return carry

        lax.fori_loop(0, R, blk, 0)
        w_wait(R - 1)

    return pl.kernel(
        body,
        mesh=plsc.VectorSubcoreMesh(core_axis_name="c", subcore_axis_name="s"),
        compiler_params=pltpu.CompilerParams(use_tc_tiling_on_sc=False,
                                             needs_layout_passes=False),
        out_type=jax.ShapeDtypeStruct((R, _D, T), jnp.float32),
        scratch_types=[
            pltpu.VMEM((T // _NW * R,), jnp.int32),
            pltpu.VMEM((4, _TB), jnp.int32),
            pltpu.VMEM((4, _TB, _D), jnp.float32),
            pltpu.VMEM((_D, T // _NW), jnp.float32),
            pltpu.VMEM((_D * 513,), jnp.float32),
            pltpu.SemaphoreType.DMA((4,)),
            pltpu.SemaphoreType.DMA,
        ],
    )


def kernel(token_ids, weight):
    T, R = token_ids.shape
    V = weight.shape[0]
    flat_ids = token_ids.reshape(-1).astype(jnp.int32)
    v_full = (V // _SW) * _SW
    wt_pad = jnp.pad(weight[v_full:].T, ((0, 0), (0, 128 - (V - v_full))))
    table_flat = _make_transpose(V)(weight.T, wt_pad)
    out3 = _make_lookup(T, R)(flat_ids, table_flat.reshape(V, _D))
    return jnp.transpose(out3, (2, 0, 1))
